# Initial kernel scaffold; baseline (speedup 1.0000x reference)
import jax
import jax.numpy as jnp
from jax.experimental import pallas as pl


def kernel(feat0, feat1, feat2, feat3, boxes_1, boxes_2):
    raise NotImplementedError("write your pallas kernel here")


# speedup vs baseline: 33.6152x; 33.6152x over previous
import functools

import jax
import jax.numpy as jnp
from jax import lax
from jax.experimental import pallas as pl
from jax.experimental.pallas import tpu as pltpu
from jax.experimental.pallas import tpu_sc as plsc

C = 192
NB = 49
NW = 32
RPW = 16
SPATIAL_SCALE = (0.25, 0.125, 0.0625, 0.03125)
HS = (128, 64, 32, 16)
LOFF = (0, 32768, 40960, 43008)
NROWS = 43520


def _build_idx_w(boxes_1, boxes_2):
    B, M, _ = boxes_1.shape
    r1 = boxes_1.reshape(B * M, 4)
    r2 = boxes_2.reshape(B * M, 4)
    N = B * M
    batch = jnp.arange(N, dtype=jnp.int32) // M
    ux1 = jnp.minimum(r1[:, 0], r2[:, 0])
    uy1 = jnp.minimum(r1[:, 1], r2[:, 1])
    ux2 = jnp.maximum(r1[:, 2], r2[:, 2])
    uy2 = jnp.maximum(r1[:, 3], r2[:, 3])
    s1 = jnp.sqrt((r1[:, 2] - r1[:, 0]) * (r1[:, 3] - r1[:, 1]))
    s2 = jnp.sqrt((r2[:, 2] - r2[:, 0]) * (r2[:, 3] - r2[:, 1]))
    s = jnp.minimum(s1, s2)
    t = jnp.floor(4.0 + jnp.log2(s / 224.0 + 1e-6))
    lvl = jnp.clip(t, 2.0, 5.0).astype(jnp.int32) - 2

    scale = jnp.take(jnp.array(SPATIAL_SCALE, jnp.float32), lvl)
    Hf = jnp.take(jnp.array(HS, jnp.float32), lvl)
    Hi = jnp.take(jnp.array(HS, jnp.int32), lvl)
    base = jnp.take(jnp.array(LOFF, jnp.int32), lvl)

    x1 = ux1 * scale
    y1 = uy1 * scale
    x2 = ux2 * scale
    y2 = uy2 * scale
    bw = jnp.maximum(x2 - x1, 1.0) / 7.0
    bh = jnp.maximum(y2 - y1, 1.0) / 7.0

    off = (jnp.arange(14, dtype=jnp.float32) + 0.5) / 2.0

    def axis_terms(lo, bsz):
        c = lo[:, None] + off[None, :] * bsz[:, None]
        valid = (c >= -1.0) & (c <= Hf[:, None])
        cc = jnp.maximum(c, 0.0)
        c0 = jnp.minimum(jnp.floor(cc), Hf[:, None] - 1.0)
        frac = jnp.where(cc >= Hf[:, None] - 1.0, 0.0, cc - c0)
        c0i = c0.astype(jnp.int32)
        c1i = jnp.minimum(c0i + 1, Hi[:, None] - 1)
        w = jnp.stack([1.0 - frac, frac], axis=-1) * valid[:, :, None]
        ii = jnp.stack([c0i, c1i], axis=-1)
        return ii, w

    yi, wy = axis_terms(y1, bh)
    xi, wx = axis_terms(x1, bw)

    yterm = base[:, None, None] + (batch[:, None, None] * Hi[:, None, None]
                                   + yi) * Hi[:, None, None]
    yt = yterm.reshape(N, 7, 2, 2)
    wyt = wy.reshape(N, 7, 2, 2)
    xt = xi.reshape(N, 7, 2, 2)
    wxt = wx.reshape(N, 7, 2, 2)
    idx = (yt[:, :, None, :, :, None, None] +
           xt[:, None, :, None, None, :, :])
    w = (wyt[:, :, None, :, :, None, None] *
         wxt[:, None, :, None, None, :, :]) * 0.25
    return (idx.reshape(N * NB, 16).astype(jnp.int32),
            w.reshape(N * NB, 16).astype(jnp.float32))


def _sc_body(idx_hbm, w_hbm, tbl_hbm, out_hbm, idx_v, w_v, buf, stage, sem):
    wid = lax.axis_index("s") * 2 + lax.axis_index("c")
    pltpu.sync_copy(idx_hbm.at[wid], idx_v)
    pltpu.sync_copy(w_hbm.at[wid], w_v)
    lanes = jnp.arange(16, dtype=jnp.int32)
    zeros16 = jnp.zeros((16,), jnp.int32)

    def roi_body(i, carry):
        def grp_body(g, carry):
            pltpu.async_copy(tbl_hbm.at[idx_v.at[i * 7 + g]], buf, sem).wait()

            def bin_body(t, carry):
                binw = i * NB + g * 7 + t
                bir = g * 7 + t
                wrow = w_v[pl.ds(binw * 16, 16)]
                wk = [wrow[k] for k in range(16)]
                for c in range(12):
                    acc = wk[0] * buf[t * 16, pl.ds(c * 16, 16)]
                    for k in range(1, 16):
                        acc = acc + wk[k] * buf[t * 16 + k, pl.ds(c * 16, 16)]
                    stage[bir, pl.ds(c * 16, 16)] = acc
                return carry

            return lax.fori_loop(0, 7, bin_body, carry)

        carry = lax.fori_loop(0, 7, grp_body, carry)
        pltpu.sync_copy(stage, out_hbm.at[wid * RPW + i])
        return carry

    lax.fori_loop(0, RPW, roi_body, 0)


def kernel(feat0, feat1, feat2, feat3, boxes_1, boxes_2):
    tbl = jnp.concatenate([f.transpose(0, 2, 3, 1).reshape(-1, C)
                           for f in (feat0, feat1, feat2, feat3)], axis=0)
    idx, w = _build_idx_w(boxes_1, boxes_2)
    idx3 = idx.reshape(NW, 112, 112)
    w3 = w.reshape(NW, RPW * NB * 16)

    f = pl.kernel(
        _sc_body,
        out_type=jax.ShapeDtypeStruct((512, NB, C), jnp.float32),
        mesh=plsc.VectorSubcoreMesh(core_axis_name="c", subcore_axis_name="s"),
        scratch_types=[
            pltpu.VMEM((112, 112), jnp.int32),
            pltpu.VMEM((RPW * NB * 16,), jnp.float32),
            pltpu.VMEM((112, C), jnp.float32),
            pltpu.VMEM((NB, C), jnp.float32),
            pltpu.SemaphoreType.DMA,
        ],
        compiler_params=pltpu.CompilerParams(use_tc_tiling_on_sc=False),
    )
    out = f(idx3, w3, tbl)
    return out.reshape(512, 7, 7, C).transpose(0, 3, 1, 2)


# speedup vs baseline: 41.4586x; 1.2333x over previous
import functools

import jax
import jax.numpy as jnp
from jax import lax
from jax.experimental import pallas as pl
from jax.experimental.pallas import tpu as pltpu
from jax.experimental.pallas import tpu_sc as plsc

C = 192
NB = 49
NW = 32
RPW = 16
SPATIAL_SCALE = (0.25, 0.125, 0.0625, 0.03125)
HS = (128, 64, 32, 16)
LOFF = (0, 32768, 40960, 43008)
NROWS = 43520


def _build_idx_w(boxes_1, boxes_2):
    B, M, _ = boxes_1.shape
    r1 = boxes_1.reshape(B * M, 4)
    r2 = boxes_2.reshape(B * M, 4)
    N = B * M
    batch = jnp.arange(N, dtype=jnp.int32) // M
    ux1 = jnp.minimum(r1[:, 0], r2[:, 0])
    uy1 = jnp.minimum(r1[:, 1], r2[:, 1])
    ux2 = jnp.maximum(r1[:, 2], r2[:, 2])
    uy2 = jnp.maximum(r1[:, 3], r2[:, 3])
    s1 = jnp.sqrt((r1[:, 2] - r1[:, 0]) * (r1[:, 3] - r1[:, 1]))
    s2 = jnp.sqrt((r2[:, 2] - r2[:, 0]) * (r2[:, 3] - r2[:, 1]))
    s = jnp.minimum(s1, s2)
    t = jnp.floor(4.0 + jnp.log2(s / 224.0 + 1e-6))
    lvl = jnp.clip(t, 2.0, 5.0).astype(jnp.int32) - 2

    scale = jnp.take(jnp.array(SPATIAL_SCALE, jnp.float32), lvl)
    Hf = jnp.take(jnp.array(HS, jnp.float32), lvl)
    Hi = jnp.take(jnp.array(HS, jnp.int32), lvl)
    base = jnp.take(jnp.array(LOFF, jnp.int32), lvl)

    x1 = ux1 * scale
    y1 = uy1 * scale
    x2 = ux2 * scale
    y2 = uy2 * scale
    bw = jnp.maximum(x2 - x1, 1.0) / 7.0
    bh = jnp.maximum(y2 - y1, 1.0) / 7.0

    off = (jnp.arange(14, dtype=jnp.float32) + 0.5) / 2.0

    def axis_terms(lo, bsz):
        c = lo[:, None] + off[None, :] * bsz[:, None]
        valid = (c >= -1.0) & (c <= Hf[:, None])
        cc = jnp.maximum(c, 0.0)
        c0 = jnp.minimum(jnp.floor(cc), Hf[:, None] - 1.0)
        frac = jnp.where(cc >= Hf[:, None] - 1.0, 0.0, cc - c0)
        c0i = c0.astype(jnp.int32)
        c1i = jnp.minimum(c0i + 1, Hi[:, None] - 1)
        w = jnp.stack([1.0 - frac, frac], axis=-1) * valid[:, :, None]
        ii = jnp.stack([c0i, c1i], axis=-1)
        return ii, w

    yi, wy = axis_terms(y1, bh)
    xi, wx = axis_terms(x1, bw)

    yterm = base[:, None, None] + (batch[:, None, None] * Hi[:, None, None]
                                   + yi) * Hi[:, None, None]
    yt = yterm.reshape(N, 7, 2, 2)
    wyt = wy.reshape(N, 7, 2, 2)
    xt = xi.reshape(N, 7, 2, 2)
    wxt = wx.reshape(N, 7, 2, 2)
    idx = (yt[:, :, None, :, :, None, None] +
           xt[:, None, :, None, None, :, :])
    w = (wyt[:, :, None, :, :, None, None] *
         wxt[:, None, :, None, None, :, :]) * 0.25
    return (idx.reshape(N * NB, 16).astype(jnp.int32),
            w.reshape(N * NB, 16).astype(jnp.float32))


def _sc_body(idx_hbm, w_hbm, tbl_hbm, out_hbm, idx_v, w_v, buf0, buf1,
             stage, sem0, sem1):
    wid = lax.axis_index("s") * 2 + lax.axis_index("c")
    pltpu.sync_copy(idx_hbm.at[wid], idx_v)
    pltpu.sync_copy(w_hbm.at[wid], w_v)

    def compute_group(g, buf):
        def bin_body(t, carry):
            binw = g * 7 + t
            wrow = w_v[pl.ds(binw * 16, 16)]
            wk = [wrow[k] for k in range(16)]
            for c in range(12):
                acc = wk[0] * buf[t * 16, pl.ds(c * 16, 16)]
                for k in range(1, 16):
                    acc = acc + wk[k] * buf[t * 16 + k, pl.ds(c * 16, 16)]
                stage[lax.rem(binw, NB), pl.ds(c * 16, 16)] = acc
            return carry

        lax.fori_loop(0, 7, bin_body, 0)

        @pl.when(lax.rem(g, 7) == 6)
        def _():
            pltpu.sync_copy(stage, out_hbm.at[wid * RPW + lax.div(g, 7)])

    pltpu.async_copy(tbl_hbm.at[idx_v.at[0]], buf0, sem0)

    def pair_body(p, carry):
        g0 = p * 2
        pltpu.async_copy(tbl_hbm.at[idx_v.at[g0 + 1]], buf1, sem1)
        pltpu.make_async_copy(tbl_hbm.at[idx_v.at[g0]], buf0, sem0).wait()
        compute_group(g0, buf0)

        @pl.when(p < 55)
        def _():
            pltpu.async_copy(tbl_hbm.at[idx_v.at[g0 + 2]], buf0, sem0)

        pltpu.make_async_copy(tbl_hbm.at[idx_v.at[g0 + 1]], buf1, sem1).wait()
        compute_group(g0 + 1, buf1)
        return carry

    lax.fori_loop(0, 56, pair_body, 0)


def kernel(feat0, feat1, feat2, feat3, boxes_1, boxes_2):
    tbl = jnp.concatenate([f.transpose(0, 2, 3, 1).reshape(-1, C)
                           for f in (feat0, feat1, feat2, feat3)], axis=0)
    idx, w = _build_idx_w(boxes_1, boxes_2)
    idx3 = idx.reshape(NW, 112, 112)
    w3 = w.reshape(NW, RPW * NB * 16)

    f = pl.kernel(
        _sc_body,
        out_type=jax.ShapeDtypeStruct((512, NB, C), jnp.float32),
        mesh=plsc.VectorSubcoreMesh(core_axis_name="c", subcore_axis_name="s"),
        scratch_types=[
            pltpu.VMEM((112, 112), jnp.int32),
            pltpu.VMEM((RPW * NB * 16,), jnp.float32),
            pltpu.VMEM((112, C), jnp.float32),
            pltpu.VMEM((112, C), jnp.float32),
            pltpu.VMEM((NB, C), jnp.float32),
            pltpu.SemaphoreType.DMA,
            pltpu.SemaphoreType.DMA,
        ],
        compiler_params=pltpu.CompilerParams(use_tc_tiling_on_sc=False),
    )
    out = f(idx3, w3, tbl)
    return out.reshape(512, 7, 7, C).transpose(0, 3, 1, 2)


# speedup vs baseline: 72.5560x; 1.7501x over previous
import functools

import jax
import jax.numpy as jnp
import numpy as np
from jax import lax
from jax.experimental import pallas as pl
from jax.experimental.pallas import tpu as pltpu
from jax.experimental.pallas import tpu_sc as plsc

C = 192
NB = 49
NW = 32
RPW = 16
SPATIAL_SCALE = (0.25, 0.125, 0.0625, 0.03125)
HS = (128, 64, 32, 16)
LOFF = (0, 32768, 40960, 43008)
NROWS = 43520


def _build_idx_w(boxes_1, boxes_2):
    B, M, _ = boxes_1.shape
    r1 = boxes_1.reshape(B * M, 4)
    r2 = boxes_2.reshape(B * M, 4)
    N = B * M
    batch = jnp.arange(N, dtype=jnp.int32) // M
    ux1 = jnp.minimum(r1[:, 0], r2[:, 0])
    uy1 = jnp.minimum(r1[:, 1], r2[:, 1])
    ux2 = jnp.maximum(r1[:, 2], r2[:, 2])
    uy2 = jnp.maximum(r1[:, 3], r2[:, 3])
    s1 = jnp.sqrt((r1[:, 2] - r1[:, 0]) * (r1[:, 3] - r1[:, 1]))
    s2 = jnp.sqrt((r2[:, 2] - r2[:, 0]) * (r2[:, 3] - r2[:, 1]))
    s = jnp.minimum(s1, s2)
    t = jnp.floor(4.0 + jnp.log2(s / 224.0 + 1e-6))
    lvl = jnp.clip(t, 2.0, 5.0).astype(jnp.int32) - 2

    scale = jnp.take(jnp.array(SPATIAL_SCALE, jnp.float32), lvl)
    Hf = jnp.take(jnp.array(HS, jnp.float32), lvl)
    Hi = jnp.take(jnp.array(HS, jnp.int32), lvl)
    base = jnp.take(jnp.array(LOFF, jnp.int32), lvl)

    x1 = ux1 * scale
    y1 = uy1 * scale
    x2 = ux2 * scale
    y2 = uy2 * scale
    bw = jnp.maximum(x2 - x1, 1.0) / 7.0
    bh = jnp.maximum(y2 - y1, 1.0) / 7.0

    off = (jnp.arange(14, dtype=jnp.float32) + 0.5) / 2.0

    def axis_terms(lo, bsz):
        c = lo[:, None] + off[None, :] * bsz[:, None]
        valid = (c >= -1.0) & (c <= Hf[:, None])
        cc = jnp.maximum(c, 0.0)
        c0 = jnp.minimum(jnp.floor(cc), Hf[:, None] - 1.0)
        frac = jnp.where(cc >= Hf[:, None] - 1.0, 0.0, cc - c0)
        c0i = c0.astype(jnp.int32)
        c1i = jnp.minimum(c0i + 1, Hi[:, None] - 1)
        w = jnp.stack([1.0 - frac, frac], axis=-1) * valid[:, :, None]
        ii = jnp.stack([c0i, c1i], axis=-1)
        return ii, w

    yi, wy = axis_terms(y1, bh)
    xi, wx = axis_terms(x1, bw)

    yterm = base[:, None, None] + (batch[:, None, None] * Hi[:, None, None]
                                   + yi) * Hi[:, None, None]
    yv = yterm.reshape(N, 28).astype(jnp.float32)
    wyv = wy.reshape(N, 28)
    xv = xi.reshape(N, 28).astype(jnp.float32)
    wxv = wx.reshape(N, 28)

    p = np.arange(784)
    ph, pw = p // 112, (p // 16) % 7
    j, a = (p // 8) % 2, (p // 4) % 2
    k, b = (p // 2) % 2, p % 2
    my = np.zeros((28, 784), np.float32)
    my[(ph * 2 + j) * 2 + a, p] = 1.0
    mx = np.zeros((28, 784), np.float32)
    mx[(pw * 2 + k) * 2 + b, p] = 1.0
    My = jnp.asarray(my)
    Mx = jnp.asarray(mx)

    idx = (yv @ My + xv @ Mx).astype(jnp.int32)
    w = (wyv @ My) * (wxv @ Mx) * 0.25
    return idx.reshape(N * NB, 16), w.reshape(N * NB, 16)


def _sc_body(idx_hbm, w_hbm, tbl_hbm, out_hbm, idx_v, w_v, buf0, buf1,
             stage, sem0, sem1):
    wid = lax.axis_index("s") * 2 + lax.axis_index("c")
    pltpu.sync_copy(idx_hbm.at[wid], idx_v)
    pltpu.sync_copy(w_hbm.at[wid], w_v)

    def compute_group(g, buf):
        def bin_body(t, carry):
            binw = g * 7 + t
            wrow = w_v[pl.ds(binw * 16, 16)]
            wk = [wrow[k] for k in range(16)]
            for c in range(12):
                acc = wk[0] * buf[t * 16, pl.ds(c * 16, 16)]
                for k in range(1, 16):
                    acc = acc + wk[k] * buf[t * 16 + k, pl.ds(c * 16, 16)]
                stage[lax.rem(binw, NB), pl.ds(c * 16, 16)] = acc
            return carry

        lax.fori_loop(0, 7, bin_body, 0)

        @pl.when(lax.rem(g, 7) == 6)
        def _():
            pltpu.sync_copy(stage, out_hbm.at[wid * RPW + lax.div(g, 7)])

    pltpu.async_copy(tbl_hbm.at[idx_v.at[0]], buf0, sem0)

    def pair_body(p, carry):
        g0 = p * 2
        pltpu.async_copy(tbl_hbm.at[idx_v.at[g0 + 1]], buf1, sem1)
        pltpu.make_async_copy(tbl_hbm.at[idx_v.at[g0]], buf0, sem0).wait()
        compute_group(g0, buf0)

        @pl.when(p < 55)
        def _():
            pltpu.async_copy(tbl_hbm.at[idx_v.at[g0 + 2]], buf0, sem0)

        pltpu.make_async_copy(tbl_hbm.at[idx_v.at[g0 + 1]], buf1, sem1).wait()
        compute_group(g0 + 1, buf1)
        return carry

    lax.fori_loop(0, 56, pair_body, 0)


def kernel(feat0, feat1, feat2, feat3, boxes_1, boxes_2):
    tbl = jnp.concatenate([f.transpose(0, 2, 3, 1).reshape(-1, C)
                           for f in (feat0, feat1, feat2, feat3)], axis=0)
    idx, w = _build_idx_w(boxes_1, boxes_2)
    idx3 = idx.reshape(NW, 112, 112)
    w3 = w.reshape(NW, RPW * NB * 16)

    f = pl.kernel(
        _sc_body,
        out_type=jax.ShapeDtypeStruct((512, NB, C), jnp.float32),
        mesh=plsc.VectorSubcoreMesh(core_axis_name="c", subcore_axis_name="s"),
        scratch_types=[
            pltpu.VMEM((112, 112), jnp.int32),
            pltpu.VMEM((RPW * NB * 16,), jnp.float32),
            pltpu.VMEM((112, C), jnp.float32),
            pltpu.VMEM((112, C), jnp.float32),
            pltpu.VMEM((NB, C), jnp.float32),
            pltpu.SemaphoreType.DMA,
            pltpu.SemaphoreType.DMA,
        ],
        compiler_params=pltpu.CompilerParams(use_tc_tiling_on_sc=False),
    )
    out = f(idx3, w3, tbl)
    return out.reshape(512, 7, 7, C).transpose(0, 3, 1, 2)


# speedup vs baseline: 72.6995x; 1.0020x over previous
import functools

import jax
import jax.numpy as jnp
import numpy as np
from jax import lax
from jax.experimental import pallas as pl
from jax.experimental.pallas import tpu as pltpu
from jax.experimental.pallas import tpu_sc as plsc

C = 192
NB = 49
NW = 32
RPW = 16
SPATIAL_SCALE = (0.25, 0.125, 0.0625, 0.03125)
HS = (128, 64, 32, 16)
LOFF = (0, 32768, 40960, 43008)
NROWS = 43520


def _build_idx_w(boxes_1, boxes_2):
    B, M, _ = boxes_1.shape
    r1 = boxes_1.reshape(B * M, 4)
    r2 = boxes_2.reshape(B * M, 4)
    N = B * M
    batch = jnp.arange(N, dtype=jnp.int32) // M
    ux1 = jnp.minimum(r1[:, 0], r2[:, 0])
    uy1 = jnp.minimum(r1[:, 1], r2[:, 1])
    ux2 = jnp.maximum(r1[:, 2], r2[:, 2])
    uy2 = jnp.maximum(r1[:, 3], r2[:, 3])
    s1 = jnp.sqrt((r1[:, 2] - r1[:, 0]) * (r1[:, 3] - r1[:, 1]))
    s2 = jnp.sqrt((r2[:, 2] - r2[:, 0]) * (r2[:, 3] - r2[:, 1]))
    s = jnp.minimum(s1, s2)
    t = jnp.floor(4.0 + jnp.log2(s / 224.0 + 1e-6))
    lvl = jnp.clip(t, 2.0, 5.0).astype(jnp.int32) - 2

    scale = jnp.take(jnp.array(SPATIAL_SCALE, jnp.float32), lvl)
    Hf = jnp.take(jnp.array(HS, jnp.float32), lvl)
    Hi = jnp.take(jnp.array(HS, jnp.int32), lvl)
    base = jnp.take(jnp.array(LOFF, jnp.int32), lvl)

    x1 = ux1 * scale
    y1 = uy1 * scale
    x2 = ux2 * scale
    y2 = uy2 * scale
    bw = jnp.maximum(x2 - x1, 1.0) / 7.0
    bh = jnp.maximum(y2 - y1, 1.0) / 7.0

    off = (jnp.arange(14, dtype=jnp.float32) + 0.5) / 2.0

    def axis_terms(lo, bsz):
        c = lo[:, None] + off[None, :] * bsz[:, None]
        valid = (c >= -1.0) & (c <= Hf[:, None])
        cc = jnp.maximum(c, 0.0)
        c0 = jnp.minimum(jnp.floor(cc), Hf[:, None] - 1.0)
        frac = jnp.where(cc >= Hf[:, None] - 1.0, 0.0, cc - c0)
        c0i = c0.astype(jnp.int32)
        c1i = jnp.minimum(c0i + 1, Hi[:, None] - 1)
        w = jnp.stack([1.0 - frac, frac], axis=-1) * valid[:, :, None]
        ii = jnp.stack([c0i, c1i], axis=-1)
        return ii, w

    yi, wy = axis_terms(y1, bh)
    xi, wx = axis_terms(x1, bw)

    yterm = base[:, None, None] + (batch[:, None, None] * Hi[:, None, None]
                                   + yi) * Hi[:, None, None]
    yv = yterm.reshape(N, 28).astype(jnp.float32)
    wyv = wy.reshape(N, 28)
    xv = xi.reshape(N, 28).astype(jnp.float32)
    wxv = wx.reshape(N, 28)

    p = np.arange(784)
    ph, pw = p // 112, (p // 16) % 7
    j, a = (p // 8) % 2, (p // 4) % 2
    k, b = (p // 2) % 2, p % 2
    my = np.zeros((28, 784), np.float32)
    my[(ph * 2 + j) * 2 + a, p] = 1.0
    mx = np.zeros((28, 784), np.float32)
    mx[(pw * 2 + k) * 2 + b, p] = 1.0
    My = jnp.asarray(my)
    Mx = jnp.asarray(mx)

    hp = functools.partial(jnp.matmul, precision=lax.Precision.HIGHEST)
    idx = (hp(yv, My) + hp(xv, Mx)).astype(jnp.int32)
    w = hp(wyv, My) * hp(wxv, Mx) * 0.25
    return idx.reshape(N * NB, 16), w.reshape(N * NB, 16)


def _sc_body(idx_hbm, w_hbm, tbl_hbm, out_hbm, idx_v, w_v, buf0, buf1,
             stage, sem0, sem1):
    wid = lax.axis_index("s") * 2 + lax.axis_index("c")
    pltpu.sync_copy(idx_hbm.at[wid], idx_v)
    pltpu.sync_copy(w_hbm.at[wid], w_v)

    def compute_group(g, buf):
        def bin_body(t, carry):
            binw = g * 7 + t
            wrow = w_v[pl.ds(binw * 16, 16)]
            wk = [wrow[k] for k in range(16)]
            for c in range(12):
                acc = wk[0] * buf[t * 16, pl.ds(c * 16, 16)]
                for k in range(1, 16):
                    acc = acc + wk[k] * buf[t * 16 + k, pl.ds(c * 16, 16)]
                stage[lax.rem(binw, NB), pl.ds(c * 16, 16)] = acc
            return carry

        lax.fori_loop(0, 7, bin_body, 0)

        @pl.when(lax.rem(g, 7) == 6)
        def _():
            pltpu.sync_copy(stage, out_hbm.at[wid * RPW + lax.div(g, 7)])

    pltpu.async_copy(tbl_hbm.at[idx_v.at[0]], buf0, sem0)

    def pair_body(p, carry):
        g0 = p * 2
        pltpu.async_copy(tbl_hbm.at[idx_v.at[g0 + 1]], buf1, sem1)
        pltpu.make_async_copy(tbl_hbm.at[idx_v.at[g0]], buf0, sem0).wait()
        compute_group(g0, buf0)

        @pl.when(p < 55)
        def _():
            pltpu.async_copy(tbl_hbm.at[idx_v.at[g0 + 2]], buf0, sem0)

        pltpu.make_async_copy(tbl_hbm.at[idx_v.at[g0 + 1]], buf1, sem1).wait()
        compute_group(g0 + 1, buf1)
        return carry

    lax.fori_loop(0, 56, pair_body, 0)


def kernel(feat0, feat1, feat2, feat3, boxes_1, boxes_2):
    tbl = jnp.concatenate([f.transpose(0, 2, 3, 1).reshape(-1, C)
                           for f in (feat0, feat1, feat2, feat3)], axis=0)
    idx, w = _build_idx_w(boxes_1, boxes_2)
    idx3 = idx.reshape(NW, 112, 112)
    w3 = w.reshape(NW, RPW * NB * 16)

    f = pl.kernel(
        _sc_body,
        out_type=jax.ShapeDtypeStruct((512, NB, C), jnp.float32),
        mesh=plsc.VectorSubcoreMesh(core_axis_name="c", subcore_axis_name="s"),
        scratch_types=[
            pltpu.VMEM((112, 112), jnp.int32),
            pltpu.VMEM((RPW * NB * 16,), jnp.float32),
            pltpu.VMEM((112, C), jnp.float32),
            pltpu.VMEM((112, C), jnp.float32),
            pltpu.VMEM((NB, C), jnp.float32),
            pltpu.SemaphoreType.DMA,
            pltpu.SemaphoreType.DMA,
        ],
        compiler_params=pltpu.CompilerParams(use_tc_tiling_on_sc=False),
    )
    out = f(idx3, w3, tbl)
    return out.reshape(512, 7, 7, C).transpose(0, 3, 1, 2)
